# baseline (speedup 1.0000x reference)
import functools

import jax
import jax.numpy as jnp
from jax import lax
from jax.experimental import pallas as pl
from jax.experimental.pallas import tpu as pltpu
from jax.experimental.pallas import tpu_sc as plsc

N_FACTORS = 32
BATCH = 16384

NC = 2
NS = 16
L = 16
NW = NC * NS
BPW = BATCH // NW
NBLK = BPW // L

_mesh = plsc.VectorSubcoreMesh(
    core_axis_name="c", subcore_axis_name="s", num_cores=NC, num_subcores=NS
)


@functools.partial(
    pl.kernel,
    out_type=jax.ShapeDtypeStruct((BATCH,), jnp.float32),
    mesh=_mesh,
    compiler_params=pltpu.CompilerParams(needs_layout_passes=False),
    scratch_types=[
        pltpu.VMEM((BPW,), jnp.int32),
        pltpu.VMEM((BPW,), jnp.int32),
        pltpu.VMEM((L, N_FACTORS), jnp.float32),
        pltpu.VMEM((L, N_FACTORS), jnp.float32),
        pltpu.VMEM((L, 1), jnp.float32),
        pltpu.VMEM((L, 1), jnp.float32),
        pltpu.VMEM((L, N_FACTORS), jnp.float32),
        pltpu.VMEM((L, N_FACTORS), jnp.float32),
        pltpu.VMEM((L, 1), jnp.float32),
        pltpu.VMEM((L, 1), jnp.float32),
        pltpu.VMEM((BPW,), jnp.float32),
        pltpu.SemaphoreType.DMA,
        pltpu.SemaphoreType.DMA,
    ],
)
def _mf_sc_kernel(user_hbm, item_hbm, uf_hbm, itf_hbm, ub_hbm, ib_hbm,
                  out_hbm, uidx_v, iidx_v, ur_a, ir_a, ubb_a, ibb_a,
                  ur_b, ir_b, ubb_b, ibb_b, out_v, sem_a, sem_b):
    wid = lax.axis_index("s") * NC + lax.axis_index("c")
    base = wid * BPW

    pltpu.sync_copy(user_hbm.at[pl.ds(base, BPW)], uidx_v)
    pltpu.sync_copy(item_hbm.at[pl.ds(base, BPW)], iidx_v)

    lanes = lax.broadcasted_iota(jnp.int32, (L,), 0)
    zeros = jnp.zeros((L,), jnp.int32)

    def fire(g, ur, ir, ubb, ibb, sem):
        u16 = uidx_v[pl.ds(g * L, L)]
        i16 = iidx_v[pl.ds(g * L, L)]
        descs = []
        for e in range(L):
            ru = lax.squeeze(lax.slice(u16, (e,), (e + 1,)), (0,))
            ri = lax.squeeze(lax.slice(i16, (e,), (e + 1,)), (0,))
            row = pl.ds(e, 1)
            descs.append(pltpu.async_copy(
                uf_hbm.at[pl.ds(ru, 1), :], ur.at[row, :], sem))
            descs.append(pltpu.async_copy(
                itf_hbm.at[pl.ds(ri, 1), :], ir.at[row, :], sem))
            descs.append(pltpu.async_copy(
                ub_hbm.at[pl.ds(ru, 1), :], ubb.at[row, :], sem))
            descs.append(pltpu.async_copy(
                ib_hbm.at[pl.ds(ri, 1), :], ibb.at[row, :], sem))
        return descs

    def compute(g, ur, ir, ubb, ibb):
        partials = []
        for e in range(L):
            u_lo = ur[e, pl.ds(0, L)]
            u_hi = ur[e, pl.ds(L, L)]
            v_lo = ir[e, pl.ds(0, L)]
            v_hi = ir[e, pl.ds(L, L)]
            partials.append(u_lo * v_lo + u_hi * v_hi)
        for s in (1, 2, 4, 8):
            cond = (lanes & s) == 0
            nxt = []
            for j in range(0, len(partials), 2):
                a, c = partials[j], partials[j + 1]
                q = jnp.where(cond, a, c)
                t = jnp.where(cond, c, a)
                nxt.append(q + jnp.take(t, lanes ^ s))
            partials = nxt
        bias = (plsc.load_gather(ubb, [lanes, zeros])
                + plsc.load_gather(ibb, [lanes, zeros]))
        out_v[pl.ds(g * L, L)] = partials[0] + bias

    def body(i, carry):
        descs_a = fire(2 * i, ur_a, ir_a, ubb_a, ibb_a, sem_a)
        descs_b = fire(2 * i + 1, ur_b, ir_b, ubb_b, ibb_b, sem_b)
        for d in descs_a:
            d.wait()
        compute(2 * i, ur_a, ir_a, ubb_a, ibb_a)
        for d in descs_b:
            d.wait()
        compute(2 * i + 1, ur_b, ir_b, ubb_b, ibb_b)
        return carry

    lax.fori_loop(0, NBLK // 2, body, 0)

    pltpu.sync_copy(out_v, out_hbm.at[pl.ds(base, BPW)])


def kernel(user, item, user_factors, item_factors, user_biases, item_biases):
    user = user.astype(jnp.int32)
    item = item.astype(jnp.int32)
    return _mf_sc_kernel(user, item, user_factors, item_factors,
                         user_biases, item_biases)


# speedup vs baseline: 1.1439x; 1.1439x over previous
import functools

import jax
import jax.numpy as jnp
from jax import lax
from jax.experimental import pallas as pl
from jax.experimental.pallas import tpu as pltpu
from jax.experimental.pallas import tpu_sc as plsc

N_FACTORS = 32
BATCH = 16384

NC = 2
NS = 16
L = 16
NW = NC * NS
BPW = BATCH // NW
ICH = 128
NCH = BPW // ICH
NBLK = BPW // L

_mesh = plsc.VectorSubcoreMesh(
    core_axis_name="c", subcore_axis_name="s", num_cores=NC, num_subcores=NS
)


@functools.partial(
    pl.kernel,
    out_type=jax.ShapeDtypeStruct((BATCH,), jnp.float32),
    mesh=_mesh,
    compiler_params=pltpu.CompilerParams(use_tc_tiling_on_sc=False,
                                         needs_layout_passes=False),
    scratch_types=[
        pltpu.VMEM((NCH, ICH), jnp.int32),
        pltpu.VMEM((NCH, ICH), jnp.int32),
        pltpu.VMEM((BPW, N_FACTORS), jnp.float32),
        pltpu.VMEM((BPW, N_FACTORS), jnp.float32),
        pltpu.VMEM((BPW,), jnp.float32),
        pltpu.VMEM((BPW,), jnp.float32),
        pltpu.VMEM((BPW,), jnp.float32),
        pltpu.SemaphoreType.DMA,
    ],
)
def _mf_sc_kernel(user_hbm, item_hbm, uf_hbm, itf_hbm, ub_hbm, ib_hbm,
                  out_hbm, uidx_v, iidx_v, uf_v, itf_v, ub_v, ib_v,
                  out_v, sem):
    wid = lax.axis_index("s") * NC + lax.axis_index("c")
    base = wid * BPW

    for j in range(NCH):
        pltpu.sync_copy(user_hbm.at[pl.ds(base + j * ICH, ICH)], uidx_v.at[j])
        pltpu.sync_copy(item_hbm.at[pl.ds(base + j * ICH, ICH)], iidx_v.at[j])

    copies = []
    for j in range(NCH):
        sl = pl.ds(j * ICH, ICH)
        copies.append(pltpu.async_copy(uf_hbm.at[uidx_v.at[j]], uf_v.at[sl], sem))
        copies.append(pltpu.async_copy(itf_hbm.at[iidx_v.at[j]], itf_v.at[sl], sem))
        copies.append(pltpu.async_copy(ub_hbm.at[uidx_v.at[j]], ub_v.at[sl], sem))
        copies.append(pltpu.async_copy(ib_hbm.at[iidx_v.at[j]], ib_v.at[sl], sem))
    for cp in copies:
        cp.wait()

    lanes = lax.broadcasted_iota(jnp.int32, (L,), 0)

    def block(i, carry):
        rows = i * L + lanes
        acc = ub_v[pl.ds(i * L, L)] + ib_v[pl.ds(i * L, L)]
        for d in range(N_FACTORS):
            cols = jnp.full((L,), d, jnp.int32)
            u = plsc.load_gather(uf_v, [rows, cols])
            v = plsc.load_gather(itf_v, [rows, cols])
            acc = acc + u * v
        out_v[pl.ds(i * L, L)] = acc
        return carry

    lax.fori_loop(0, NBLK, block, 0)

    pltpu.sync_copy(out_v, out_hbm.at[pl.ds(base, BPW)])


def kernel(user, item, user_factors, item_factors, user_biases, item_biases):
    user = user.astype(jnp.int32)
    item = item.astype(jnp.int32)
    return _mf_sc_kernel(user, item, user_factors, item_factors,
                         user_biases.reshape(-1), item_biases.reshape(-1))


# speedup vs baseline: 1.1632x; 1.0168x over previous
import functools

import jax
import jax.numpy as jnp
from jax import lax
from jax.experimental import pallas as pl
from jax.experimental.pallas import tpu as pltpu
from jax.experimental.pallas import tpu_sc as plsc

N_FACTORS = 32
BATCH = 16384

NC = 2
NS = 16
L = 16
NW = NC * NS
BPW = BATCH // NW
ICH = 128
NCH = BPW // ICH
NBLK = BPW // L

_mesh = plsc.VectorSubcoreMesh(
    core_axis_name="c", subcore_axis_name="s", num_cores=NC, num_subcores=NS
)


@functools.partial(
    pl.kernel,
    out_type=jax.ShapeDtypeStruct((BATCH,), jnp.float32),
    mesh=_mesh,
    compiler_params=pltpu.CompilerParams(use_tc_tiling_on_sc=False),
    scratch_types=[
        pltpu.VMEM((NCH, ICH), jnp.int32),
        pltpu.VMEM((NCH, ICH), jnp.int32),
        pltpu.VMEM((BPW, N_FACTORS), jnp.float32),
        pltpu.VMEM((BPW, N_FACTORS), jnp.float32),
        pltpu.VMEM((BPW,), jnp.float32),
        pltpu.VMEM((BPW,), jnp.float32),
        pltpu.VMEM((BPW,), jnp.float32),
        pltpu.SemaphoreType.DMA,
    ],
)
def _mf_sc_kernel(user_hbm, item_hbm, uf_hbm, itf_hbm, ub_hbm, ib_hbm,
                  out_hbm, uidx_v, iidx_v, uf_v, itf_v, ub_v, ib_v,
                  out_v, sem):
    wid = lax.axis_index("s") * NC + lax.axis_index("c")
    base = wid * BPW

    for j in range(NCH):
        pltpu.sync_copy(user_hbm.at[pl.ds(base + j * ICH, ICH)], uidx_v.at[j])
        pltpu.sync_copy(item_hbm.at[pl.ds(base + j * ICH, ICH)], iidx_v.at[j])

    copies = []
    for j in range(NCH):
        sl = pl.ds(j * ICH, ICH)
        copies.append(pltpu.async_copy(uf_hbm.at[uidx_v.at[j]], uf_v.at[sl], sem))
        copies.append(pltpu.async_copy(itf_hbm.at[iidx_v.at[j]], itf_v.at[sl], sem))
        copies.append(pltpu.async_copy(ub_hbm.at[uidx_v.at[j]], ub_v.at[sl], sem))
        copies.append(pltpu.async_copy(ib_hbm.at[iidx_v.at[j]], ib_v.at[sl], sem))
    for cp in copies:
        cp.wait()

    lanes = lax.broadcasted_iota(jnp.int32, (L,), 0)

    def block(i, carry):
        b0 = i * L
        partials = []
        for e in range(L):
            u_lo = uf_v[b0 + e, pl.ds(0, L)]
            u_hi = uf_v[b0 + e, pl.ds(L, L)]
            v_lo = itf_v[b0 + e, pl.ds(0, L)]
            v_hi = itf_v[b0 + e, pl.ds(L, L)]
            partials.append(u_lo * v_lo + u_hi * v_hi)
        for s in (1, 2, 4, 8):
            cond = (lanes & s) == 0
            nxt = []
            for j in range(0, len(partials), 2):
                a, c = partials[j], partials[j + 1]
                q = jnp.where(cond, a, c)
                t = jnp.where(cond, c, a)
                nxt.append(q + jnp.take(t, lanes ^ s))
            partials = nxt
        out_v[pl.ds(b0, L)] = (partials[0] + ub_v[pl.ds(b0, L)]
                               + ib_v[pl.ds(b0, L)])
        return carry

    lax.fori_loop(0, NBLK, block, 0)

    pltpu.sync_copy(out_v, out_hbm.at[pl.ds(base, BPW)])


def kernel(user, item, user_factors, item_factors, user_biases, item_biases):
    user = user.astype(jnp.int32)
    item = item.astype(jnp.int32)
    return _mf_sc_kernel(user, item, user_factors, item_factors,
                         user_biases.reshape(-1), item_biases.reshape(-1))
